# baseline (speedup 1.0000x reference)
import functools

import jax
import jax.numpy as jnp
from jax import lax
from jax.experimental import pallas as pl
from jax.experimental.pallas import tpu as pltpu
from jax.experimental.pallas import tpu_sc as plsc

_N_BINS = 15


def _stage1_body(logits_ref, labels_ref, conf_ref, acc_ref):
    x = logits_ref[...]
    y = labels_ref[...]
    mx = jnp.max(x, axis=0, keepdims=True)
    my = jnp.max(y, axis=0, keepdims=True)
    hit = (x >= mx) & (y >= my)
    acc_ref[...] = jnp.any(hit, axis=0).astype(jnp.float32)
    conf_ref[...] = jax.nn.sigmoid(mx).reshape(mx.shape[1])


def _stage1(logits_t, labels_t, block_cols, half_cols, off_blocks):
    n_classes, _ = logits_t.shape
    n_steps = half_cols // block_cols
    return pl.pallas_call(
        _stage1_body,
        grid=(n_steps,),
        in_specs=[
            pl.BlockSpec((n_classes, block_cols), lambda i: (0, i + off_blocks)),
            pl.BlockSpec((n_classes, block_cols), lambda i: (0, i + off_blocks)),
        ],
        out_specs=[
            pl.BlockSpec((block_cols,), lambda i: (i,)),
            pl.BlockSpec((block_cols,), lambda i: (i,)),
        ],
        out_shape=[
            jax.ShapeDtypeStruct((half_cols,), jnp.float32),
            jax.ShapeDtypeStruct((half_cols,), jnp.float32),
        ],
        compiler_params=pltpu.CompilerParams(
            dimension_semantics=("arbitrary",),
        ),
    )(logits_t, labels_t)


def _stage2(conf, acc):
    n_samples = conf.shape[0]
    info = plsc.get_sparse_core_info()
    nc, ns, nl = info.num_cores, info.num_subcores, info.num_lanes
    nw = nc * ns
    chunk = n_samples // nw
    n_vec = chunk // nl
    mesh = plsc.VectorSubcoreMesh(core_axis_name="c", subcore_axis_name="s")

    @functools.partial(
        pl.kernel,
        mesh=mesh,
        out_type=jax.ShapeDtypeStruct((nw, 3 * _N_BINS * nl), jnp.float32),
        scratch_types=[
            pltpu.VMEM((chunk,), jnp.float32),
            pltpu.VMEM((chunk,), jnp.float32),
            pltpu.VMEM((3 * _N_BINS * nl,), jnp.float32),
            pltpu.SemaphoreType.DMA,
            pltpu.SemaphoreType.DMA,
        ],
    )
    def sc_hist(conf_hbm, acc_hbm, out_hbm, conf_v, acc_v, h_all, sem1, sem2):
        wid = lax.axis_index("s") * nc + lax.axis_index("c")
        base = wid * chunk
        cp1 = pltpu.async_copy(conf_hbm.at[pl.ds(base, chunk)], conf_v, sem1)
        cp2 = pltpu.async_copy(acc_hbm.at[pl.ds(base, chunk)], acc_v, sem2)
        cp1.wait()
        cp2.wait()
        zero = jnp.zeros((nl,), jnp.float32)
        one = jnp.ones((nl,), jnp.float32)

        def body(i, carry):
            cnts, confs, accs = carry
            off = i * nl
            v = conf_v[pl.ds(off, nl)]
            a = acc_v[pl.ds(off, nl)]
            b0 = jnp.minimum((v * 15.0).astype(jnp.int32), 14)
            bf = b0.astype(jnp.float32)
            low = bf / 15.0
            up = (bf + 1.0) / 15.0
            bi = jnp.where(v <= low, b0 - 1, jnp.where(v > up, b0 + 1, b0))
            cnts = tuple(
                c + jnp.where(bi == b, one, zero) for b, c in enumerate(cnts))
            confs = tuple(
                c + jnp.where(bi == b, v, zero) for b, c in enumerate(confs))
            accs = tuple(
                c + jnp.where(bi == b, a, zero) for b, c in enumerate(accs))
            return cnts, confs, accs

        init = (tuple(zero for _ in range(_N_BINS)),
                tuple(zero for _ in range(_N_BINS)),
                tuple(zero for _ in range(_N_BINS)))
        cnts, confs, accs = lax.fori_loop(0, n_vec, body, init)
        nb = _N_BINS * nl
        for b in range(_N_BINS):
            h_all[pl.ds(b * nl, nl)] = cnts[b]
            h_all[pl.ds(nb + b * nl, nl)] = confs[b]
            h_all[pl.ds(2 * nb + b * nl, nl)] = accs[b]
        pltpu.sync_copy(h_all, out_hbm.at[wid])

    return sc_hist(conf, acc)


def _stage3_body(hist1_ref, hist2_ref, out_ref, *, n_workers, n_lanes,
                 n_samples):
    h = hist1_ref[...] + hist2_ref[...]
    row = jnp.sum(h, axis=0, keepdims=True)
    nb = _N_BINS * n_lanes

    def per_stat(lo):
        return jnp.concatenate(
            [jnp.sum(row[:, lo + b * n_lanes:lo + (b + 1) * n_lanes], axis=1,
                     keepdims=True) for b in range(_N_BINS)], axis=1)

    cnt = per_stat(0)
    sc = per_stat(nb)
    sa = per_stat(2 * nb)
    prop = cnt / jnp.float32(n_samples)
    safe = jnp.maximum(cnt, 1.0)
    contrib = jnp.abs(sc / safe - sa / safe) * prop
    contrib = jnp.where(cnt > 0.0, contrib, 0.0)
    out_ref[...] = jnp.sum(contrib, axis=1, keepdims=True)


def _stage3(hist1, hist2, n_workers, n_lanes, n_samples):
    body = functools.partial(_stage3_body, n_workers=n_workers,
                             n_lanes=n_lanes, n_samples=n_samples)
    return pl.pallas_call(
        body,
        out_shape=jax.ShapeDtypeStruct((1, 1), jnp.float32),
    )(hist1, hist2)


@jax.jit
def _ece(logits, labels):
    n_samples, _ = logits.shape
    xt, yt = logits.T, labels.T
    if n_samples % 4096 == 0:
        block_cols, half = 2048, n_samples // 2
        conf1, acc1 = _stage1(xt, yt, block_cols, half, 0)
        hist1 = _stage2(conf1, acc1)
        conf2, acc2 = _stage1(xt, yt, block_cols, half, half // block_cols)
        hist2 = _stage2(conf2, acc2)
    else:
        conf1, acc1 = _stage1(xt, yt, n_samples, n_samples, 0)
        hist1 = _stage2(conf1, acc1)
        hist2 = jnp.zeros_like(hist1)
    n_workers, statbinlanes = hist1.shape
    out = _stage3(hist1, hist2, n_workers, statbinlanes // (3 * _N_BINS),
                  n_samples)
    return out.reshape(1)


def kernel(logits, labels):
    return _ece(logits, labels)


# speedup vs baseline: 1.0195x; 1.0195x over previous
import functools

import jax
import jax.numpy as jnp
from jax import lax
from jax.experimental import pallas as pl
from jax.experimental.pallas import tpu as pltpu
from jax.experimental.pallas import tpu_sc as plsc

_N_BINS = 15


def _stage1_body(logits_ref, labels_ref, conf_ref, acc_ref):
    x = logits_ref[...]
    y = labels_ref[...]
    mx = jnp.max(x, axis=0, keepdims=True)
    my = jnp.max(y, axis=0, keepdims=True)
    hit = (x >= mx) & (y >= my)
    acc_ref[...] = jnp.any(hit, axis=0).astype(jnp.float32)
    conf_ref[...] = jax.nn.sigmoid(mx).reshape(mx.shape[1])


def _stage1(logits_t, labels_t, block_cols):
    n_classes, n_samples = logits_t.shape
    n_steps = n_samples // block_cols
    return pl.pallas_call(
        _stage1_body,
        grid=(n_steps,),
        in_specs=[
            pl.BlockSpec((n_classes, block_cols), lambda i: (0, i)),
            pl.BlockSpec((n_classes, block_cols), lambda i: (0, i)),
        ],
        out_specs=[
            pl.BlockSpec((block_cols,), lambda i: (i,)),
            pl.BlockSpec((block_cols,), lambda i: (i,)),
        ],
        out_shape=[
            jax.ShapeDtypeStruct((n_samples,), jnp.float32),
            jax.ShapeDtypeStruct((n_samples,), jnp.float32),
        ],
        compiler_params=pltpu.CompilerParams(
            dimension_semantics=("arbitrary",),
        ),
    )(logits_t, labels_t)


def _stage2(conf, acc):
    n_samples = conf.shape[0]
    info = plsc.get_sparse_core_info()
    nc, ns, nl = info.num_cores, info.num_subcores, info.num_lanes
    nw = nc * ns
    chunk = n_samples // nw
    n_vec = chunk // nl
    mesh = plsc.VectorSubcoreMesh(core_axis_name="c", subcore_axis_name="s")

    @functools.partial(
        pl.kernel,
        mesh=mesh,
        out_type=jax.ShapeDtypeStruct((nw, 3 * _N_BINS * nl), jnp.float32),
        scratch_types=[
            pltpu.VMEM((chunk,), jnp.float32),
            pltpu.VMEM((chunk,), jnp.float32),
            pltpu.VMEM((3 * _N_BINS * nl,), jnp.float32),
            pltpu.SemaphoreType.DMA,
            pltpu.SemaphoreType.DMA,
        ],
    )
    def sc_hist(conf_hbm, acc_hbm, out_hbm, conf_v, acc_v, h_all, sem1, sem2):
        wid = lax.axis_index("s") * nc + lax.axis_index("c")
        base = wid * chunk
        cp1 = pltpu.async_copy(conf_hbm.at[pl.ds(base, chunk)], conf_v, sem1)
        cp2 = pltpu.async_copy(acc_hbm.at[pl.ds(base, chunk)], acc_v, sem2)
        cp1.wait()
        cp2.wait()
        zero = jnp.zeros((nl,), jnp.float32)
        one = jnp.ones((nl,), jnp.float32)

        def body(i, carry):
            cnts, confs, accs = carry
            off = i * nl
            v = conf_v[pl.ds(off, nl)]
            a = acc_v[pl.ds(off, nl)]
            b0 = jnp.minimum((v * 15.0).astype(jnp.int32), 14)
            bf = b0.astype(jnp.float32)
            low = bf / 15.0
            up = (bf + 1.0) / 15.0
            bi = jnp.where(v <= low, b0 - 1, jnp.where(v > up, b0 + 1, b0))
            cnts = tuple(
                c + jnp.where(bi == b, one, zero) for b, c in enumerate(cnts))
            confs = tuple(
                c + jnp.where(bi == b, v, zero) for b, c in enumerate(confs))
            accs = tuple(
                c + jnp.where(bi == b, a, zero) for b, c in enumerate(accs))
            return cnts, confs, accs

        init = (tuple(zero for _ in range(_N_BINS)),
                tuple(zero for _ in range(_N_BINS)),
                tuple(zero for _ in range(_N_BINS)))
        cnts, confs, accs = lax.fori_loop(0, n_vec, body, init)
        nb = _N_BINS * nl
        for b in range(_N_BINS):
            h_all[pl.ds(b * nl, nl)] = cnts[b]
            h_all[pl.ds(nb + b * nl, nl)] = confs[b]
            h_all[pl.ds(2 * nb + b * nl, nl)] = accs[b]
        pltpu.sync_copy(h_all, out_hbm.at[wid])

    return sc_hist(conf, acc)


def _stage3_body(hist_ref, out_ref, *, n_workers, n_lanes, n_samples):
    h = hist_ref[...]
    row = jnp.sum(h, axis=0, keepdims=True)
    nb = _N_BINS * n_lanes

    def per_stat(lo):
        return jnp.concatenate(
            [jnp.sum(row[:, lo + b * n_lanes:lo + (b + 1) * n_lanes], axis=1,
                     keepdims=True) for b in range(_N_BINS)], axis=1)

    cnt = per_stat(0)
    sc = per_stat(nb)
    sa = per_stat(2 * nb)
    prop = cnt / jnp.float32(n_samples)
    safe = jnp.maximum(cnt, 1.0)
    contrib = jnp.abs(sc / safe - sa / safe) * prop
    contrib = jnp.where(cnt > 0.0, contrib, 0.0)
    out_ref[...] = jnp.sum(contrib, axis=1, keepdims=True)


def _stage3(hist2d, n_workers, n_lanes, n_samples):
    body = functools.partial(_stage3_body, n_workers=n_workers,
                             n_lanes=n_lanes, n_samples=n_samples)
    return pl.pallas_call(
        body,
        out_shape=jax.ShapeDtypeStruct((1, 1), jnp.float32),
    )(hist2d)


@jax.jit
def _ece(logits, labels):
    n_samples, _ = logits.shape
    block_cols = 2048 if n_samples % 2048 == 0 else n_samples
    conf, acc = _stage1(logits.T, labels.T, block_cols)
    hist = _stage2(conf, acc)
    n_workers, statbinlanes = hist.shape
    out = _stage3(hist, n_workers, statbinlanes // (3 * _N_BINS), n_samples)
    return out.reshape(1)


def kernel(logits, labels):
    return _ece(logits, labels)


# speedup vs baseline: 1.0227x; 1.0032x over previous
import functools

import jax
import jax.numpy as jnp
from jax import lax
from jax.experimental import pallas as pl
from jax.experimental.pallas import tpu as pltpu
from jax.experimental.pallas import tpu_sc as plsc

_N_BINS = 15


def _stage1_body(logits_ref, labels_ref, conf_ref, acc_ref):
    x = logits_ref[...]
    y = labels_ref[...]
    mx = jnp.max(x, axis=0, keepdims=True)
    my = jnp.max(y, axis=0, keepdims=True)
    hit = (x >= mx) & (y >= my)
    acc_ref[...] = jnp.any(hit, axis=0).astype(jnp.float32)
    conf_ref[...] = jax.nn.sigmoid(mx).reshape(mx.shape[1])


def _stage1(logits_t, labels_t, block_cols):
    n_classes, n_samples = logits_t.shape
    n_steps = n_samples // block_cols
    return pl.pallas_call(
        _stage1_body,
        grid=(n_steps,),
        in_specs=[
            pl.BlockSpec((n_classes, block_cols), lambda i: (0, i)),
            pl.BlockSpec((n_classes, block_cols), lambda i: (0, i)),
        ],
        out_specs=[
            pl.BlockSpec((block_cols,), lambda i: (i,)),
            pl.BlockSpec((block_cols,), lambda i: (i,)),
        ],
        out_shape=[
            jax.ShapeDtypeStruct((n_samples,), jnp.float32),
            jax.ShapeDtypeStruct((n_samples,), jnp.float32),
        ],
        compiler_params=pltpu.CompilerParams(
            dimension_semantics=("arbitrary",),
        ),
    )(logits_t, labels_t)


def _stage2(conf, acc):
    n_samples = conf.shape[0]
    info = plsc.get_sparse_core_info()
    nc, ns, nl = info.num_cores, info.num_subcores, info.num_lanes
    nw = nc * ns
    chunk = n_samples // nw
    n_vec = chunk // nl
    mesh = plsc.VectorSubcoreMesh(core_axis_name="c", subcore_axis_name="s")

    @functools.partial(
        pl.kernel,
        mesh=mesh,
        out_type=jax.ShapeDtypeStruct((nw, 2 * _N_BINS * nl), jnp.float32),
        scratch_types=[
            pltpu.VMEM((chunk,), jnp.float32),
            pltpu.VMEM((chunk,), jnp.float32),
            pltpu.VMEM((2 * _N_BINS * nl,), jnp.float32),
            pltpu.SemaphoreType.DMA,
            pltpu.SemaphoreType.DMA,
        ],
    )
    def sc_hist(conf_hbm, acc_hbm, out_hbm, conf_v, acc_v, h_all, sem1, sem2):
        wid = lax.axis_index("s") * nc + lax.axis_index("c")
        base = wid * chunk
        cp1 = pltpu.async_copy(conf_hbm.at[pl.ds(base, chunk)], conf_v, sem1)
        cp2 = pltpu.async_copy(acc_hbm.at[pl.ds(base, chunk)], acc_v, sem2)
        cp1.wait()
        cp2.wait()
        zero = jnp.zeros((nl,), jnp.float32)
        one = jnp.ones((nl,), jnp.float32)

        def body(i, carry):
            packs, confs = carry
            off = i * nl
            v = conf_v[pl.ds(off, nl)]
            a = acc_v[pl.ds(off, nl)]
            base = one + a * 4096.0
            b0 = jnp.minimum((v * 15.0).astype(jnp.int32), 14)
            bf = b0.astype(jnp.float32)
            low = bf / 15.0
            up = (bf + 1.0) / 15.0
            bi = jnp.where(v <= low, b0 - 1, jnp.where(v > up, b0 + 1, b0))
            packs = tuple(
                c + jnp.where(bi == b, base, zero) for b, c in enumerate(packs))
            confs = tuple(
                c + jnp.where(bi == b, v, zero) for b, c in enumerate(confs))
            return packs, confs

        init = (tuple(zero for _ in range(_N_BINS)),
                tuple(zero for _ in range(_N_BINS)))
        packs, confs = lax.fori_loop(0, n_vec, body, init)
        nb = _N_BINS * nl
        for b in range(_N_BINS):
            h_all[pl.ds(b * nl, nl)] = packs[b]
            h_all[pl.ds(nb + b * nl, nl)] = confs[b]
        pltpu.sync_copy(h_all, out_hbm.at[wid])

    return sc_hist(conf, acc)


def _stage3_body(hist_ref, out_ref, *, n_workers, n_lanes, n_samples):
    h = hist_ref[...]
    nb = _N_BINS * n_lanes
    pack = h[:, 0:nb]
    sa_m = jnp.floor(pack * (1.0 / 4096.0))
    cnt_m = pack - 4096.0 * sa_m
    row_cnt = jnp.sum(cnt_m, axis=0, keepdims=True)
    row_sa = jnp.sum(sa_m, axis=0, keepdims=True)
    row_sc = jnp.sum(h[:, nb:2 * nb], axis=0, keepdims=True)

    def per_bin(row):
        return jnp.concatenate(
            [jnp.sum(row[:, b * n_lanes:(b + 1) * n_lanes], axis=1,
                     keepdims=True) for b in range(_N_BINS)], axis=1)

    cnt = per_bin(row_cnt)
    sc = per_bin(row_sc)
    sa = per_bin(row_sa)
    prop = cnt / jnp.float32(n_samples)
    safe = jnp.maximum(cnt, 1.0)
    contrib = jnp.abs(sc / safe - sa / safe) * prop
    contrib = jnp.where(cnt > 0.0, contrib, 0.0)
    out_ref[...] = jnp.sum(contrib, axis=1, keepdims=True)


def _stage3(hist2d, n_workers, n_lanes, n_samples):
    body = functools.partial(_stage3_body, n_workers=n_workers,
                             n_lanes=n_lanes, n_samples=n_samples)
    return pl.pallas_call(
        body,
        out_shape=jax.ShapeDtypeStruct((1, 1), jnp.float32),
    )(hist2d)


@jax.jit
def _ece(logits, labels):
    n_samples, _ = logits.shape
    block_cols = 2048 if n_samples % 2048 == 0 else n_samples
    conf, acc = _stage1(logits.T, labels.T, block_cols)
    hist = _stage2(conf, acc)
    n_workers, statbinlanes = hist.shape
    out = _stage3(hist, n_workers, statbinlanes // (2 * _N_BINS), n_samples)
    return out.reshape(1)


def kernel(logits, labels):
    return _ece(logits, labels)
